# baseline (speedup 1.0000x reference)
import functools

import jax
import jax.numpy as jnp
from jax import lax
from jax.experimental import pallas as pl
from jax.experimental.pallas import tpu as pltpu
from jax.experimental.pallas import tpu_sc as plsc

_B, _L, _M, _N, _C = 2, 2048, 16, 16, 64
_H = _W = 64
_HW = _H * _W
_NC, _NS = 2, 16
_NW = _NC * _NS
_A = _B * _L
_APW = _A // _NW


def _weights_tc(feats_t):
    def body(f_ref, o_ref):
        o_ref[...] = jnp.sum(f_ref[...], axis=2)

    return pl.pallas_call(
        body,
        grid=(_B,),
        in_specs=[pl.BlockSpec((1, _N, _C, _L), lambda b: (b, 0, 0, 0))],
        out_specs=pl.BlockSpec((1, _N, _L), lambda b: (b, 0, 0)),
        out_shape=jax.ShapeDtypeStruct((_B, _N, _L), jnp.float32),
    )(feats_t)


def _transpose_tc(hist2d):
    pb = 256

    def body(x_ref, o_ref):
        o_ref[0] = x_ref[...].T

    return pl.pallas_call(
        body,
        grid=(_B, _HW // pb),
        in_specs=[pl.BlockSpec((_L, pb), lambda b, j: (b, j))],
        out_specs=pl.BlockSpec((1, pb, _L), lambda b, j: (b, j, 0)),
        out_shape=jax.ShapeDtypeStruct((_B, _HW, _L), jnp.float32),
    )(hist2d)


def _make_sc_hist():
    mesh = plsc.VectorSubcoreMesh(core_axis_name="c", subcore_axis_name="s")

    @functools.partial(
        pl.kernel,
        mesh=mesh,
        compiler_params=pltpu.CompilerParams(needs_layout_passes=False),
        out_type=jax.ShapeDtypeStruct((_A, _HW), jnp.float32),
        scratch_types=[
            pltpu.VMEM((_L,), jnp.int32),
            pltpu.VMEM((_L,), jnp.int32),
            pltpu.VMEM((_L,), jnp.int32),
            pltpu.VMEM((_L,), jnp.int32),
            pltpu.VMEM((_M, _APW), jnp.int32),
            pltpu.VMEM((_N, _APW), jnp.int32),
            pltpu.VMEM((_N, _APW), jnp.float32),
            pltpu.VMEM((_HW,), jnp.float32),
            pltpu.VMEM((_HW,), jnp.float32),
            pltpu.VMEM((_M * _N,), jnp.int32),
            pltpu.VMEM((_M * _N,), jnp.int32),
            pltpu.VMEM((48,), jnp.int32),
            pltpu.SemaphoreType.DMA,
            pltpu.SemaphoreType.DMA,
        ],
    )
    def sc_hist(vsy_h, vsx_h, vdy_h, vdx_h, isrc_h, idst_h, wts_h, out_h,
                vsy_v, vsx_v, vdy_v, vdx_v, isrc_v, idst_v, wts_v,
                hist0_v, hist1_v, bins0_v, bins1_v, src_s, sem0, sem1):
        wid = lax.axis_index("c") * _NS + lax.axis_index("s")
        batch = wid // (_NW // _B)
        lbase = (wid % (_NW // _B)) * _APW
        base = wid * _APW
        pltpu.sync_copy(vsy_h.at[batch], vsy_v)
        pltpu.sync_copy(vsx_h.at[batch], vsx_v)
        pltpu.sync_copy(vdy_h.at[batch], vdy_v)
        pltpu.sync_copy(vdx_h.at[batch], vdx_v)
        pltpu.sync_copy(isrc_h.at[batch, :, pl.ds(lbase, _APW)], isrc_v)
        pltpu.sync_copy(idst_h.at[batch, :, pl.ds(lbase, _APW)], idst_v)
        pltpu.sync_copy(wts_h.at[batch, :, pl.ds(lbase, _APW)], wts_v)

        zv = jnp.zeros((16,), jnp.float32)
        iota = lax.iota(jnp.int32, 16)
        sems = (sem0, sem1)
        hists = (hist0_v, hist1_v)
        binsv = (bins0_v, bins1_v)

        for hb in hists:
            for t in range(_HW // 16):
                hb[pl.ds(t * 16, 16)] = zv

        def pair_body(g, carry):
            for p in range(2):
                a = g * 2 + p
                hb = hists[p]
                bb = binsv[p]
                sem = sems[p]

                @pl.when(g > 0)
                def _undo():
                    pltpu.make_async_copy(hb, out_h.at[base], sem).wait()
                    wn = -plsc.load_gather(wts_v, [iota, jnp.full((16,), a - 2, jnp.int32)])
                    for i in range(_M):
                        fb = bb[pl.ds(i * 16, 16)]
                        plsc.addupdate_scatter(hb, [fb], wn)

                col = jnp.full((16,), a, jnp.int32)
                si = plsc.load_gather(isrc_v, [iota, col])
                di = plsc.load_gather(idst_v, [iota, col])
                w = plsc.load_gather(wts_v, [iota, col])
                src_s[pl.ds(16, 16)] = plsc.load_gather(vsy_v, [si])
                src_s[pl.ds(32, 16)] = plsc.load_gather(vsx_v, [si])
                dy = plsc.load_gather(vdy_v, [di])
                dx = plsc.load_gather(vdx_v, [di])
                for i in range(_M):
                    syi = plsc.load_gather(src_s, [jnp.full((16,), 16 + i, jnp.int32)])
                    sxi = plsc.load_gather(src_s, [jnp.full((16,), 32 + i, jnp.int32)])
                    by = lax.shift_right_arithmetic(dy - syi, 2)
                    bx = lax.shift_right_arithmetic(dx - sxi, 2)
                    flat = by * _W + bx + (_H // 2 * _W + _W // 2)
                    bb[pl.ds(i * 16, 16)] = flat
                    plsc.addupdate_scatter(hb, [flat], w)
                pltpu.async_copy(hb, out_h.at[base + a], sem)
            return carry

        lax.fori_loop(0, _APW // 2, pair_body, 0)
        for p in range(2):
            pltpu.make_async_copy(hists[p], out_h.at[base], sems[p]).wait()

    return sc_hist


_sc_hist = _make_sc_hist()


def kernel(feats_src_dst, voxels_src, voxels_dst, idxs_src, idxs_dst):
    b, l, n, c = feats_src_dst.shape
    wts_t = _weights_tc(feats_src_dst.transpose(0, 2, 3, 1))
    vsy = voxels_src[..., 0]
    vsx = voxels_src[..., 1]
    vdy = voxels_dst[..., 0]
    vdx = voxels_dst[..., 1]
    isrc_t = idxs_src.transpose(0, 2, 1)
    idst_t = idxs_dst.transpose(0, 2, 1)
    hist2d = _sc_hist(vsy, vsx, vdy, vdx, isrc_t, idst_t, wts_t)
    out_t = _transpose_tc(hist2d)
    return out_t.reshape(_B, _H, _W, _L).transpose(0, 3, 1, 2)


# speedup vs baseline: 1.0686x; 1.0686x over previous
import functools

import jax
import jax.numpy as jnp
from jax import lax
from jax.experimental import pallas as pl
from jax.experimental.pallas import tpu as pltpu
from jax.experimental.pallas import tpu_sc as plsc

_B, _L, _M, _N, _C = 2, 2048, 16, 16, 64
_H = _W = 64
_HW = _H * _W
_NC, _NS = 2, 16
_NW = _NC * _NS
_A = _B * _L
_APW = _A // _NW


def _weights_tc(feats_t):
    def body(f_ref, o_ref):
        o_ref[...] = jnp.sum(f_ref[...], axis=2)

    return pl.pallas_call(
        body,
        grid=(_B,),
        in_specs=[pl.BlockSpec((1, _N, _C, _L), lambda b: (b, 0, 0, 0))],
        out_specs=pl.BlockSpec((1, _N, _L), lambda b: (b, 0, 0)),
        out_shape=jax.ShapeDtypeStruct((_B, _N, _L), jnp.float32),
    )(feats_t)


def _transpose_tc(hist2d):
    pb = 1024

    def body(x_ref, o_ref):
        o_ref[0] = x_ref[...].T

    return pl.pallas_call(
        body,
        grid=(_B, _HW // pb),
        in_specs=[pl.BlockSpec((_L, pb), lambda b, j: (b, j))],
        out_specs=pl.BlockSpec((1, pb, _L), lambda b, j: (b, j, 0)),
        out_shape=jax.ShapeDtypeStruct((_B, _HW, _L), jnp.float32),
    )(hist2d)


def _make_sc_hist():
    mesh = plsc.VectorSubcoreMesh(core_axis_name="c", subcore_axis_name="s")

    @functools.partial(
        pl.kernel,
        mesh=mesh,
        compiler_params=pltpu.CompilerParams(needs_layout_passes=False),
        out_type=jax.ShapeDtypeStruct((_A, _HW), jnp.float32),
        scratch_types=[
            pltpu.VMEM((_L,), jnp.int32),
            pltpu.VMEM((_L,), jnp.int32),
            pltpu.VMEM((_L,), jnp.int32),
            pltpu.VMEM((_L,), jnp.int32),
            pltpu.VMEM((_M, _APW), jnp.int32),
            pltpu.VMEM((_N, _APW), jnp.int32),
            pltpu.VMEM((_N, _APW), jnp.float32),
            pltpu.VMEM((_HW,), jnp.float32),
            pltpu.VMEM((_HW,), jnp.float32),
            pltpu.VMEM((_M * _N,), jnp.int32),
            pltpu.VMEM((_M * _N,), jnp.int32),
            pltpu.VMEM((48,), jnp.int32),
            pltpu.SemaphoreType.DMA,
            pltpu.SemaphoreType.DMA,
        ],
    )
    def sc_hist(vsy_h, vsx_h, vdy_h, vdx_h, isrc_h, idst_h, wts_h, out_h,
                vsy_v, vsx_v, vdy_v, vdx_v, isrc_v, idst_v, wts_v,
                hist0_v, hist1_v, bins0_v, bins1_v, src_s, sem0, sem1):
        wid = lax.axis_index("c") * _NS + lax.axis_index("s")
        batch = wid // (_NW // _B)
        lbase = (wid % (_NW // _B)) * _APW
        base = wid * _APW
        pltpu.sync_copy(vsy_h.at[batch], vsy_v)
        pltpu.sync_copy(vsx_h.at[batch], vsx_v)
        pltpu.sync_copy(vdy_h.at[batch], vdy_v)
        pltpu.sync_copy(vdx_h.at[batch], vdx_v)
        pltpu.sync_copy(isrc_h.at[batch, :, pl.ds(lbase, _APW)], isrc_v)
        pltpu.sync_copy(idst_h.at[batch, :, pl.ds(lbase, _APW)], idst_v)
        pltpu.sync_copy(wts_h.at[batch, :, pl.ds(lbase, _APW)], wts_v)

        zv = jnp.zeros((16,), jnp.float32)
        iota = lax.iota(jnp.int32, 16)
        sems = (sem0, sem1)
        hists = (hist0_v, hist1_v)
        binsv = (bins0_v, bins1_v)

        for hb in hists:
            for t in range(_HW // 16):
                hb[pl.ds(t * 16, 16)] = zv

        def pair_body(g, carry):
            for p in range(2):
                a = g * 2 + p
                hb = hists[p]
                bb = binsv[p]
                sem = sems[p]

                @pl.when(g > 0)
                def _undo():
                    pltpu.make_async_copy(hb, out_h.at[base], sem).wait()
                    wn = -plsc.load_gather(wts_v, [iota, jnp.full((16,), a - 2, jnp.int32)])
                    for i in range(_M):
                        fb = bb[pl.ds(i * 16, 16)]
                        plsc.addupdate_scatter(hb, [fb], wn)

                col = jnp.full((16,), a, jnp.int32)
                si = plsc.load_gather(isrc_v, [iota, col])
                di = plsc.load_gather(idst_v, [iota, col])
                w = plsc.load_gather(wts_v, [iota, col])
                src_s[pl.ds(16, 16)] = plsc.load_gather(vsy_v, [si])
                src_s[pl.ds(32, 16)] = plsc.load_gather(vsx_v, [si])
                dy = plsc.load_gather(vdy_v, [di])
                dx = plsc.load_gather(vdx_v, [di])
                for i in range(_M):
                    syi = plsc.load_gather(src_s, [jnp.full((16,), 16 + i, jnp.int32)])
                    sxi = plsc.load_gather(src_s, [jnp.full((16,), 32 + i, jnp.int32)])
                    by = lax.shift_right_arithmetic(dy - syi, 2)
                    bx = lax.shift_right_arithmetic(dx - sxi, 2)
                    flat = by * _W + bx + (_H // 2 * _W + _W // 2)
                    bb[pl.ds(i * 16, 16)] = flat
                    plsc.addupdate_scatter(hb, [flat], w)
                pltpu.async_copy(hb, out_h.at[base + a], sem)
            return carry

        lax.fori_loop(0, _APW // 2, pair_body, 0)
        for p in range(2):
            pltpu.make_async_copy(hists[p], out_h.at[base], sems[p]).wait()

    return sc_hist


_sc_hist = _make_sc_hist()


def kernel(feats_src_dst, voxels_src, voxels_dst, idxs_src, idxs_dst):
    b, l, n, c = feats_src_dst.shape
    wts_t = _weights_tc(feats_src_dst.transpose(0, 2, 3, 1))
    vsy = voxels_src[..., 0]
    vsx = voxels_src[..., 1]
    vdy = voxels_dst[..., 0]
    vdx = voxels_dst[..., 1]
    isrc_t = idxs_src.transpose(0, 2, 1)
    idst_t = idxs_dst.transpose(0, 2, 1)
    hist2d = _sc_hist(vsy, vsx, vdy, vdx, isrc_t, idst_t, wts_t)
    out_t = _transpose_tc(hist2d)
    return out_t.reshape(_B, _H, _W, _L).transpose(0, 3, 1, 2)
